# baseline (speedup 1.0000x reference)
import functools
import jax
import jax.numpy as jnp
from jax import lax
from jax.experimental import pallas as pl
from jax.experimental.pallas import tpu as pltpu
from jax.experimental.pallas import tpu_sc as plsc

N = 10000
E = 160000
R = 21
D = 256
H = 256
L = 3
DH = 128

NPAD = 10240
BN = 256
BE = 256
NBLK = 656
EP = NBLK * BE
CHUNK = 128
CNT_ROWS = 212992
TRASH_SEG = N * R
TRASH_NODE = N

NC = 2
NS = 16
NW = NC * NS

_mesh = functools.partial(
    plsc.VectorSubcoreMesh, core_axis_name="c", subcore_axis_name="s")


NBUF_G = 3
NBUF_S = 2


def _gather_body(src_hbm, h_hbm, g_hbm, idxbuf, rows, sem_i, sem_g, sem_s):
    cid = lax.axis_index("c")
    sid = lax.axis_index("s")
    wid = sid * NC + cid
    per_tile = EP // NW
    base0 = wid * per_tile
    nchunk = per_tile // CHUNK
    ngrp = nchunk // NBUF_G
    tail = nchunk - ngrp * NBUF_G

    def grp(gi, carry):
        i0 = gi * NBUF_G
        for b in range(NBUF_G):
            base = base0 + (i0 + b) * CHUNK
            pltpu.async_copy(src_hbm.at[pl.ds(base, CHUNK)], idxbuf.at[b], sem_i)
        for b in range(NBUF_G):
            pltpu.make_async_copy(src_hbm.at[pl.ds(base0, CHUNK)], idxbuf.at[b], sem_i).wait()
        for b in range(NBUF_G):
            pltpu.async_copy(h_hbm.at[idxbuf.at[b]], rows.at[b], sem_g.at[b])
        for b in range(NBUF_G):
            base = base0 + (i0 + b) * CHUNK
            pltpu.make_async_copy(h_hbm.at[idxbuf.at[b]], rows.at[b], sem_g.at[b]).wait()
            pltpu.async_copy(rows.at[b], g_hbm.at[pl.ds(base, CHUNK)], sem_s)
        for b in range(NBUF_G):
            base = base0 + (i0 + b) * CHUNK
            pltpu.make_async_copy(rows.at[b], g_hbm.at[pl.ds(base, CHUNK)], sem_s).wait()
        return carry

    lax.fori_loop(0, ngrp, grp, 0)
    for t in range(tail):
        i = ngrp * NBUF_G + t
        base = base0 + i * CHUNK
        pltpu.sync_copy(src_hbm.at[pl.ds(base, CHUNK)], idxbuf.at[0])
        pltpu.async_copy(h_hbm.at[idxbuf.at[0]], rows.at[0], sem_g.at[0]).wait()
        pltpu.sync_copy(rows.at[0], g_hbm.at[pl.ds(base, CHUNK)])


def _sc_gather(src_p, table):
    w = table.shape[1]
    return pl.kernel(
        _gather_body,
        mesh=_mesh(),
        out_type=jax.ShapeDtypeStruct((EP, w), jnp.float32),
        scratch_types=[
            pltpu.VMEM((NBUF_G, CHUNK), jnp.int32),
            pltpu.VMEM((NBUF_G, CHUNK, w), jnp.float32),
            pltpu.SemaphoreType.DMA,
            pltpu.SemaphoreType.DMA((NBUF_G,)),
            pltpu.SemaphoreType.DMA,
        ],
    )(src_p, table)


def _scatter_body(dst_hbm, y_hbm, zeros_hbm, conv_hbm, idxbuf, rows, acc,
                  sem_i, sem_y):
    cid = lax.axis_index("c")
    sid = lax.axis_index("s")
    stripe = NPAD // NS
    r0 = sid * stripe
    pltpu.sync_copy(zeros_hbm.at[pl.ds(r0, stripe)], acc.at[pl.ds(r0, stripe)])
    plsc.subcore_barrier()

    per_tile = EP // NS
    base0 = sid * per_tile
    nchunk = per_tile // CHUNK
    ngrp = nchunk // 2

    def fire(c, b):
        base = base0 + c * CHUNK
        pltpu.async_copy(dst_hbm.at[pl.ds(base, CHUNK)], idxbuf.at[b], sem_i.at[b])
        pltpu.async_copy(y_hbm.at[pl.ds(cid * EP + base, CHUNK)], rows.at[b], sem_y.at[b])

    def drain_add(b):
        pltpu.make_async_copy(dst_hbm.at[pl.ds(base0, CHUNK)], idxbuf.at[b], sem_i.at[b]).wait()
        pltpu.make_async_copy(y_hbm.at[pl.ds(base0, CHUNK)], rows.at[b], sem_y.at[b]).wait()
        pltpu.sync_copy(rows.at[b], acc.at[idxbuf.at[b]], add=True)

    fire(0, 0)

    def grp(p, carry):
        c0 = 2 * p
        fire(c0 + 1, 1)
        drain_add(0)

        @pl.when(p < ngrp - 1)
        def _():
            fire(c0 + 2, 0)

        drain_add(1)
        return carry

    lax.fori_loop(0, ngrp, grp, 0)

    plsc.subcore_barrier()
    pltpu.sync_copy(acc.at[pl.ds(r0, stripe)],
                    conv_hbm.at[pl.ds(cid * NPAD + r0, stripe)])


def _sc_scatter(dst_p, y_flat, zeros_nod):
    w = y_flat.shape[1]
    return pl.kernel(
        _scatter_body,
        mesh=_mesh(),
        out_type=jax.ShapeDtypeStruct((2 * NPAD, w), jnp.float32),
        scratch_types=[
            pltpu.VMEM((NBUF_S, CHUNK), jnp.int32),
            pltpu.VMEM((NBUF_S, CHUNK, w), jnp.float32),
            pltpu.VMEM_SHARED((NPAD, w), jnp.float32),
            pltpu.SemaphoreType.DMA((NBUF_S,)),
            pltpu.SemaphoreType.DMA((NBUF_S,)),
        ],
    )(dst_p, y_flat, zeros_nod)


def _inv_body(cnt_ref, oh_ref, valid_ref, o_ref):
    cnt = jnp.sum(cnt_ref[...] * oh_ref[...], axis=-1)
    o_ref[0, 0, :] = valid_ref[0, 0, :] / jnp.clip(cnt, 1.0, None)


def _tc_inv(cntrows, oh, valid3):
    return pl.pallas_call(
        _inv_body,
        grid=(NBLK,),
        in_specs=[
            pl.BlockSpec((BE, DH), lambda i: (i, 0)),
            pl.BlockSpec((BE, DH), lambda i: (i, 0)),
            pl.BlockSpec((1, 1, BE), lambda i: (i, 0, 0)),
        ],
        out_specs=pl.BlockSpec((1, 1, BE), lambda i: (i, 0, 0)),
        out_shape=jax.ShapeDtypeStruct((NBLK, 1, BE), jnp.float32),
    )(cntrows, oh, valid3)


def _msg_body(blkrel_ref, g_ref, inv_ref, wr_ref, y_ref):
    z = g_ref[...] * inv_ref[0, 0, :][:, None]
    y = jnp.dot(z, wr_ref[0], preferred_element_type=jnp.float32)
    y_ref[0] = y[:, :DH]
    y_ref[1] = y[:, DH:]


def _tc_msg(blkrel, g, inv3, Wr):
    grid_spec = pltpu.PrefetchScalarGridSpec(
        num_scalar_prefetch=1,
        grid=(NBLK,),
        in_specs=[
            pl.BlockSpec((BE, D), lambda i, br: (i, 0)),
            pl.BlockSpec((1, 1, BE), lambda i, br: (i, 0, 0)),
            pl.BlockSpec((1, D, H), lambda i, br: (br[i], 0, 0)),
        ],
        out_specs=pl.BlockSpec((2, BE, DH), lambda i, br: (0, i, 0)),
    )
    return pl.pallas_call(
        _msg_body,
        grid_spec=grid_spec,
        out_shape=jax.ShapeDtypeStruct((2, EP, DH), jnp.float32),
    )(blkrel, g, inv3, Wr)


def _node_body(c0_ref, c1_ref, h_ref, wo_ref, b_ref, g_ref, bb_ref, o_ref):
    h = h_ref[...]
    acc = jnp.dot(h, wo_ref[...], preferred_element_type=jnp.float32)
    acc += b_ref[...]
    acc += jnp.concatenate([c0_ref[...], c1_ref[...]], axis=-1)
    t = jnp.maximum(acc, 0.0) + h
    mu = jnp.mean(t, axis=-1, keepdims=True)
    var = jnp.mean(t * t, axis=-1, keepdims=True) - mu * mu
    o_ref[...] = (t - mu) * lax.rsqrt(var + 1e-5) * g_ref[...] + bb_ref[...]


def _tc_node(conv, h, Wo, b, g, bb):
    return pl.pallas_call(
        _node_body,
        grid=(NPAD // BN,),
        in_specs=[
            pl.BlockSpec((BN, DH), lambda i: (i, 0)),
            pl.BlockSpec((BN, DH), lambda i: (NPAD // BN + i, 0)),
            pl.BlockSpec((BN, H), lambda i: (i, 0)),
            pl.BlockSpec((H, H), lambda i: (0, 0)),
            pl.BlockSpec((1, H), lambda i: (0, 0)),
            pl.BlockSpec((1, H), lambda i: (0, 0)),
            pl.BlockSpec((1, H), lambda i: (0, 0)),
        ],
        out_specs=pl.BlockSpec((BN, H), lambda i: (i, 0)),
        out_shape=jax.ShapeDtypeStruct((NPAD, H), jnp.float32),
    )(conv, conv, h, Wo, b, g, bb)


def _mlp_body(h_ref, w1_ref, b1_ref, w2_ref, b2_ref, o_ref):
    h = h_ref[...]
    t = jnp.maximum(jnp.dot(h, w1_ref[...], preferred_element_type=jnp.float32) + b1_ref[...], 0.0)
    o_ref[...] = jnp.dot(t, w2_ref[...], preferred_element_type=jnp.float32) + b2_ref[...]


def _tc_mlp(h, W1, b1, W2, b2):
    return pl.pallas_call(
        _mlp_body,
        grid=(NPAD // BN,),
        in_specs=[
            pl.BlockSpec((BN, H), lambda i: (i, 0)),
            pl.BlockSpec((H, DH), lambda i: (0, 0)),
            pl.BlockSpec((1, DH), lambda i: (0, 0)),
            pl.BlockSpec((DH, 1), lambda i: (0, 0)),
            pl.BlockSpec((1, 1), lambda i: (0, 0)),
        ],
        out_specs=pl.BlockSpec((BN, 1), lambda i: (i, 0)),
        out_shape=jax.ShapeDtypeStruct((NPAD, 1), jnp.float32),
    )(h, W1, b1, W2, b2)


def kernel(x, edge_index, edge_attr, edge_type, Wrel, Wroot, bconv, ln_g, ln_b, W1, b1, W2, b2):
    i32 = jnp.int32
    src = edge_index[0]
    dst = edge_index[1]
    et = edge_type

    rids = jnp.arange(R, dtype=i32)
    cnt_r = jnp.sum((et[:, None] == rids[None, :]).astype(i32), axis=0)
    order = jnp.argsort(et)
    start = jnp.concatenate([jnp.zeros((1,), i32), jnp.cumsum(cnt_r)]).astype(i32)
    nblk_r = (cnt_r + BE - 1) // BE
    pstart = jnp.concatenate([jnp.zeros((1,), i32),
                              jnp.cumsum(nblk_r * BE)]).astype(i32)

    j = jnp.arange(EP, dtype=i32)
    r_of = jnp.sum((j[:, None] >= pstart[None, :R]).astype(i32), axis=1) - 1
    jl = j - jnp.take(pstart, r_of)
    valid = jl < jnp.take(cnt_r, r_of)
    sidx = jnp.clip(jnp.take(start, r_of) + jl, 0, E - 1)
    eid = jnp.take(order, sidx).astype(i32)
    src_g = jnp.take(src, eid)
    dst_g = jnp.take(dst, eid)
    et_g = jnp.take(et, eid)
    src_p = jnp.where(valid, src_g, 0).astype(i32)
    dst_p = jnp.where(valid, dst_g, TRASH_NODE).astype(i32)
    etp = jnp.where(valid, et_g, 0).astype(i32)
    valid3 = valid.astype(jnp.float32).reshape(NBLK, 1, BE)
    blkrel = r_of.reshape(NBLK, BE)[:, 0].astype(i32)

    zeros_nod = jnp.zeros((NPAD, DH), jnp.float32)

    oh = (etp[:, None] == jnp.arange(DH, dtype=i32)[None, :]).astype(jnp.float32)
    ycnt = jnp.concatenate([oh, jnp.zeros((EP, DH), jnp.float32)], axis=0)
    cnttab = _sc_scatter(dst_p, ycnt, zeros_nod)[:NPAD]
    cntrows = _sc_gather(dst_p, cnttab)
    inv3 = _tc_inv(cntrows, oh, valid3)

    h = jnp.pad(x, ((0, NPAD - N), (0, 0)))
    for l in range(L):
        g = _sc_gather(src_p, h)
        y = _tc_msg(blkrel, g, inv3, Wrel[l])
        conv = _sc_scatter(dst_p, y.reshape(2 * EP, DH), zeros_nod)
        h = _tc_node(conv, h, Wroot[l], bconv[l][None, :],
                     ln_g[l][None, :], ln_b[l][None, :])
    out = _tc_mlp(h, W1, b1[None, :], W2, b2[None, :])
    return out[:N]


# speedup vs baseline: 1.0565x; 1.0565x over previous
import functools
import jax
import jax.numpy as jnp
from jax import lax
from jax.experimental import pallas as pl
from jax.experimental.pallas import tpu as pltpu
from jax.experimental.pallas import tpu_sc as plsc

N = 10000
E = 160000
R = 21
D = 256
H = 256
L = 3
DH = 128

NPAD = 10240
BN = 256
BE = 256
NBLK = 656
EP = NBLK * BE
CHUNK = 128
CNT_ROWS = 212992
TRASH_SEG = N * R
TRASH_NODE = N

NC = 2
NS = 16
NW = NC * NS

_mesh = functools.partial(
    plsc.VectorSubcoreMesh, core_axis_name="c", subcore_axis_name="s")


NBUF_G = 3
NBUF_S = 2


def _gather_body(src_hbm, h_hbm, g_hbm, idxbuf, rows, sem_i, sem_g, sem_s):
    cid = lax.axis_index("c")
    sid = lax.axis_index("s")
    wid = sid * NC + cid
    per_tile = EP // NW
    base0 = wid * per_tile
    nchunk = per_tile // CHUNK
    ngrp = nchunk // NBUF_G
    tail = nchunk - ngrp * NBUF_G

    def grp(gi, carry):
        i0 = gi * NBUF_G
        for b in range(NBUF_G):
            base = base0 + (i0 + b) * CHUNK
            pltpu.async_copy(src_hbm.at[pl.ds(base, CHUNK)], idxbuf.at[b], sem_i)
        for b in range(NBUF_G):
            pltpu.make_async_copy(src_hbm.at[pl.ds(base0, CHUNK)], idxbuf.at[b], sem_i).wait()
        for b in range(NBUF_G):
            pltpu.async_copy(h_hbm.at[idxbuf.at[b]], rows.at[b], sem_g.at[b])
        for b in range(NBUF_G):
            base = base0 + (i0 + b) * CHUNK
            pltpu.make_async_copy(h_hbm.at[idxbuf.at[b]], rows.at[b], sem_g.at[b]).wait()
            pltpu.async_copy(rows.at[b], g_hbm.at[pl.ds(base, CHUNK)], sem_s)
        for b in range(NBUF_G):
            base = base0 + (i0 + b) * CHUNK
            pltpu.make_async_copy(rows.at[b], g_hbm.at[pl.ds(base, CHUNK)], sem_s).wait()
        return carry

    lax.fori_loop(0, ngrp, grp, 0)
    for t in range(tail):
        i = ngrp * NBUF_G + t
        base = base0 + i * CHUNK
        pltpu.sync_copy(src_hbm.at[pl.ds(base, CHUNK)], idxbuf.at[0])
        pltpu.async_copy(h_hbm.at[idxbuf.at[0]], rows.at[0], sem_g.at[0]).wait()
        pltpu.sync_copy(rows.at[0], g_hbm.at[pl.ds(base, CHUNK)])


def _sc_gather(src_p, table):
    w = table.shape[1]
    return pl.kernel(
        _gather_body,
        mesh=_mesh(),
        out_type=jax.ShapeDtypeStruct((EP, w), jnp.float32),
        scratch_types=[
            pltpu.VMEM((NBUF_G, CHUNK), jnp.int32),
            pltpu.VMEM((NBUF_G, CHUNK, w), jnp.float32),
            pltpu.SemaphoreType.DMA,
            pltpu.SemaphoreType.DMA((NBUF_G,)),
            pltpu.SemaphoreType.DMA,
        ],
    )(src_p, table)


def _scatter_body(dst_hbm, y_hbm, zeros_hbm, conv_hbm, idxbuf, rows, acc,
                  sem_i, sem_y):
    cid = lax.axis_index("c")
    sid = lax.axis_index("s")
    stripe = NPAD // NS
    r0 = sid * stripe
    pltpu.sync_copy(zeros_hbm.at[pl.ds(r0, stripe)], acc.at[pl.ds(r0, stripe)])
    plsc.subcore_barrier()

    per_tile = EP // NS
    base0 = sid * per_tile
    nchunk = per_tile // CHUNK
    ngrp = nchunk // 2

    def fire(c, b):
        base = base0 + c * CHUNK
        pltpu.async_copy(dst_hbm.at[pl.ds(base, CHUNK)], idxbuf.at[b], sem_i.at[b])
        pltpu.async_copy(y_hbm.at[pl.ds(cid * EP + base, CHUNK)], rows.at[b], sem_y.at[b])

    def drain_add(b):
        pltpu.make_async_copy(dst_hbm.at[pl.ds(base0, CHUNK)], idxbuf.at[b], sem_i.at[b]).wait()
        pltpu.make_async_copy(y_hbm.at[pl.ds(base0, CHUNK)], rows.at[b], sem_y.at[b]).wait()
        pltpu.sync_copy(rows.at[b], acc.at[idxbuf.at[b]], add=True)

    fire(0, 0)

    def grp(p, carry):
        c0 = 2 * p
        fire(c0 + 1, 1)
        drain_add(0)

        @pl.when(p < ngrp - 1)
        def _():
            fire(c0 + 2, 0)

        drain_add(1)
        return carry

    lax.fori_loop(0, ngrp, grp, 0)

    plsc.subcore_barrier()
    pltpu.sync_copy(acc.at[pl.ds(r0, stripe)],
                    conv_hbm.at[pl.ds(cid * NPAD + r0, stripe)])


def _sc_scatter(dst_p, y_flat, zeros_nod):
    w = y_flat.shape[1]
    return pl.kernel(
        _scatter_body,
        mesh=_mesh(),
        out_type=jax.ShapeDtypeStruct((2 * NPAD, w), jnp.float32),
        scratch_types=[
            pltpu.VMEM((NBUF_S, CHUNK), jnp.int32),
            pltpu.VMEM((NBUF_S, CHUNK, w), jnp.float32),
            pltpu.VMEM_SHARED((NPAD, w), jnp.float32),
            pltpu.SemaphoreType.DMA((NBUF_S,)),
            pltpu.SemaphoreType.DMA((NBUF_S,)),
        ],
    )(dst_p, y_flat, zeros_nod)


def _cscatter_body(dst_hbm, oh_hbm, zeros_hbm, out_hbm, idxbuf, rows, acc,
                   sem_i, sem_y):
    cid = lax.axis_index("c")
    sid = lax.axis_index("s")
    stripe = NPAD // NS
    r0 = sid * stripe
    pltpu.sync_copy(zeros_hbm.at[pl.ds(r0, stripe)], acc.at[pl.ds(r0, stripe)])
    plsc.subcore_barrier()

    per_tile = EP // NW
    base0 = cid * (EP // NC) + sid * per_tile
    nchunk = per_tile // CHUNK

    def fire(c, b):
        base = base0 + c * CHUNK
        pltpu.async_copy(dst_hbm.at[pl.ds(base, CHUNK)], idxbuf.at[b], sem_i.at[b])
        pltpu.async_copy(oh_hbm.at[pl.ds(base, CHUNK)], rows.at[b], sem_y.at[b])

    def drain_add(b):
        pltpu.make_async_copy(dst_hbm.at[pl.ds(base0, CHUNK)], idxbuf.at[b], sem_i.at[b]).wait()
        pltpu.make_async_copy(oh_hbm.at[pl.ds(base0, CHUNK)], rows.at[b], sem_y.at[b]).wait()
        pltpu.sync_copy(rows.at[b], acc.at[idxbuf.at[b]], add=True)

    fire(0, 0)

    def grp(p, carry):
        c0 = 2 * p
        fire(c0 + 1, 1)
        drain_add(0)
        fire(c0 + 2, 0)
        drain_add(1)
        return carry

    lax.fori_loop(0, (nchunk - 1) // 2, grp, 0)
    drain_add(0)

    plsc.subcore_barrier()
    pltpu.sync_copy(acc.at[pl.ds(r0, stripe)],
                    out_hbm.at[pl.ds(cid * NPAD + r0, stripe)])


def _sc_cscatter(dst_p, oh, zeros_nod):
    return pl.kernel(
        _cscatter_body,
        mesh=_mesh(),
        out_type=jax.ShapeDtypeStruct((2 * NPAD, DH), jnp.float32),
        scratch_types=[
            pltpu.VMEM((NBUF_S, CHUNK), jnp.int32),
            pltpu.VMEM((NBUF_S, CHUNK, DH), jnp.float32),
            pltpu.VMEM_SHARED((NPAD, DH), jnp.float32),
            pltpu.SemaphoreType.DMA((NBUF_S,)),
            pltpu.SemaphoreType.DMA((NBUF_S,)),
        ],
    )(dst_p, oh, zeros_nod)


def _inv_body(cnt_ref, oh_ref, valid_ref, o_ref):
    cnt = jnp.sum(cnt_ref[...] * oh_ref[...], axis=-1)
    o_ref[0, 0, :] = valid_ref[0, 0, :] / jnp.clip(cnt, 1.0, None)


def _tc_inv(cntrows, oh, valid3):
    return pl.pallas_call(
        _inv_body,
        grid=(NBLK,),
        in_specs=[
            pl.BlockSpec((BE, DH), lambda i: (i, 0)),
            pl.BlockSpec((BE, DH), lambda i: (i, 0)),
            pl.BlockSpec((1, 1, BE), lambda i: (i, 0, 0)),
        ],
        out_specs=pl.BlockSpec((1, 1, BE), lambda i: (i, 0, 0)),
        out_shape=jax.ShapeDtypeStruct((NBLK, 1, BE), jnp.float32),
    )(cntrows, oh, valid3)


def _msg_body(blkrel_ref, g_ref, inv_ref, wr_ref, y_ref):
    z = g_ref[...] * inv_ref[0, 0, :][:, None]
    y = jnp.dot(z, wr_ref[0], preferred_element_type=jnp.float32)
    y_ref[0] = y[:, :DH]
    y_ref[1] = y[:, DH:]


def _tc_msg(blkrel, g, inv3, Wr):
    grid_spec = pltpu.PrefetchScalarGridSpec(
        num_scalar_prefetch=1,
        grid=(NBLK,),
        in_specs=[
            pl.BlockSpec((BE, D), lambda i, br: (i, 0)),
            pl.BlockSpec((1, 1, BE), lambda i, br: (i, 0, 0)),
            pl.BlockSpec((1, D, H), lambda i, br: (br[i], 0, 0)),
        ],
        out_specs=pl.BlockSpec((2, BE, DH), lambda i, br: (0, i, 0)),
    )
    return pl.pallas_call(
        _msg_body,
        grid_spec=grid_spec,
        out_shape=jax.ShapeDtypeStruct((2, EP, DH), jnp.float32),
    )(blkrel, g, inv3, Wr)


def _node_body(c0_ref, c1_ref, h_ref, wo_ref, b_ref, g_ref, bb_ref, o_ref):
    h = h_ref[...]
    acc = jnp.dot(h, wo_ref[...], preferred_element_type=jnp.float32)
    acc += b_ref[...]
    acc += jnp.concatenate([c0_ref[...], c1_ref[...]], axis=-1)
    t = jnp.maximum(acc, 0.0) + h
    mu = jnp.mean(t, axis=-1, keepdims=True)
    var = jnp.mean(t * t, axis=-1, keepdims=True) - mu * mu
    o_ref[...] = (t - mu) * lax.rsqrt(var + 1e-5) * g_ref[...] + bb_ref[...]


def _tc_node(conv, h, Wo, b, g, bb):
    return pl.pallas_call(
        _node_body,
        grid=(NPAD // BN,),
        in_specs=[
            pl.BlockSpec((BN, DH), lambda i: (i, 0)),
            pl.BlockSpec((BN, DH), lambda i: (NPAD // BN + i, 0)),
            pl.BlockSpec((BN, H), lambda i: (i, 0)),
            pl.BlockSpec((H, H), lambda i: (0, 0)),
            pl.BlockSpec((1, H), lambda i: (0, 0)),
            pl.BlockSpec((1, H), lambda i: (0, 0)),
            pl.BlockSpec((1, H), lambda i: (0, 0)),
        ],
        out_specs=pl.BlockSpec((BN, H), lambda i: (i, 0)),
        out_shape=jax.ShapeDtypeStruct((NPAD, H), jnp.float32),
    )(conv, conv, h, Wo, b, g, bb)


def _mlp_body(h_ref, w1_ref, b1_ref, w2_ref, b2_ref, o_ref):
    h = h_ref[...]
    t = jnp.maximum(jnp.dot(h, w1_ref[...], preferred_element_type=jnp.float32) + b1_ref[...], 0.0)
    o_ref[...] = jnp.dot(t, w2_ref[...], preferred_element_type=jnp.float32) + b2_ref[...]


def _tc_mlp(h, W1, b1, W2, b2):
    return pl.pallas_call(
        _mlp_body,
        grid=(NPAD // BN,),
        in_specs=[
            pl.BlockSpec((BN, H), lambda i: (i, 0)),
            pl.BlockSpec((H, DH), lambda i: (0, 0)),
            pl.BlockSpec((1, DH), lambda i: (0, 0)),
            pl.BlockSpec((DH, 1), lambda i: (0, 0)),
            pl.BlockSpec((1, 1), lambda i: (0, 0)),
        ],
        out_specs=pl.BlockSpec((BN, 1), lambda i: (i, 0)),
        out_shape=jax.ShapeDtypeStruct((NPAD, 1), jnp.float32),
    )(h, W1, b1, W2, b2)


def kernel(x, edge_index, edge_attr, edge_type, Wrel, Wroot, bconv, ln_g, ln_b, W1, b1, W2, b2):
    i32 = jnp.int32
    src = edge_index[0]
    dst = edge_index[1]
    et = edge_type

    rids = jnp.arange(R, dtype=i32)
    cnt_r = jnp.sum((et[:, None] == rids[None, :]).astype(i32), axis=0)
    order = jnp.argsort(et)
    start = jnp.concatenate([jnp.zeros((1,), i32), jnp.cumsum(cnt_r)]).astype(i32)
    nblk_r = (cnt_r + BE - 1) // BE
    pstart = jnp.concatenate([jnp.zeros((1,), i32),
                              jnp.cumsum(nblk_r * BE)]).astype(i32)

    j = jnp.arange(EP, dtype=i32)
    r_of = jnp.sum((j[:, None] >= pstart[None, :R]).astype(i32), axis=1) - 1
    jl = j - jnp.take(pstart, r_of)
    valid = jl < jnp.take(cnt_r, r_of)
    sidx = jnp.clip(jnp.take(start, r_of) + jl, 0, E - 1)
    eid = jnp.take(order, sidx).astype(i32)
    src_g = jnp.take(src, eid)
    dst_g = jnp.take(dst, eid)
    et_g = jnp.take(et, eid)
    src_p = jnp.where(valid, src_g, 0).astype(i32)
    dst_p = jnp.where(valid, dst_g, TRASH_NODE).astype(i32)
    etp = jnp.where(valid, et_g, 0).astype(i32)
    valid3 = valid.astype(jnp.float32).reshape(NBLK, 1, BE)
    blkrel = r_of.reshape(NBLK, BE)[:, 0].astype(i32)

    zeros_nod = jnp.zeros((NPAD, DH), jnp.float32)

    oh = (etp[:, None] == jnp.arange(DH, dtype=i32)[None, :]).astype(jnp.float32)
    parts = _sc_cscatter(dst_p, oh, zeros_nod)
    cnttab = parts[:NPAD] + parts[NPAD:]
    cntrows = _sc_gather(dst_p, cnttab)
    inv3 = _tc_inv(cntrows, oh, valid3)

    h = jnp.pad(x, ((0, NPAD - N), (0, 0)))
    for l in range(L):
        g = _sc_gather(src_p, h)
        y = _tc_msg(blkrel, g, inv3, Wrel[l])
        conv = _sc_scatter(dst_p, y.reshape(2 * EP, DH), zeros_nod)
        h = _tc_node(conv, h, Wroot[l], bconv[l][None, :],
                     ln_g[l][None, :], ln_b[l][None, :])
    out = _tc_mlp(h, W1, b1[None, :], W2, b2[None, :])
    return out[:N]
